# baseline (speedup 1.0000x reference)
import functools

import jax
import jax.numpy as jnp
from jax import lax
from jax.experimental import pallas as pl
from jax.experimental.pallas import tpu as pltpu
from jax.experimental.pallas import tpu_sc as plsc

_VOCAB = 100000
_EMBD = 128
_CTX = 10
_HID = 512
_BN = 4096
_NB = (_VOCAB + _BN - 1) // _BN


def _sc_gather(idx, emb):
    n = idx.shape[0]
    mesh = plsc.VectorSubcoreMesh(core_axis_name="c", subcore_axis_name="s")

    @functools.partial(
        pl.kernel,
        out_type=jax.ShapeDtypeStruct((n, _EMBD), jnp.float32),
        mesh=mesh,
        scratch_types=[
            pltpu.VMEM((n,), jnp.int32),
            pltpu.VMEM((n, _EMBD), jnp.float32),
            pltpu.SemaphoreType.DMA,
        ],
    )
    def k(idx_hbm, emb_hbm, out_hbm, idx_v, rows_v, sem):
        c = lax.axis_index("c")
        s = lax.axis_index("s")

        @pl.when(jnp.logical_and(c == 0, s == 0))
        def _():
            pltpu.sync_copy(idx_hbm, idx_v)
            pltpu.async_copy(emb_hbm.at[idx_v], rows_v, sem).wait()
            pltpu.sync_copy(rows_v, out_hbm)

    return k(idx, emb)


def _mlp_body(e_ref, w1_ref, b1_ref, w2_ref, b2_ref, out_ref, hid_s, out_s, sm):
    p = pl.program_id(0)
    i = pl.program_id(1)

    @pl.when(jnp.logical_and(p == 0, i == 0))
    def _():
        h = jnp.dot(e_ref[...], w1_ref[...], preferred_element_type=jnp.float32)
        hid_s[...] = jnp.maximum(h + b1_ref[...], 0.0)
        sm[0] = -jnp.inf
        sm[1] = 0.0

    @pl.when(p == 0)
    def _():
        blk = jnp.dot(hid_s[...], w2_ref[...],
                      preferred_element_type=jnp.float32) + b2_ref[...]
        col = i * _BN + lax.broadcasted_iota(jnp.int32, (1, _BN), 1)
        valid = col < _VOCAB
        blkm = jnp.where(valid, blk, -jnp.inf)
        out_s[:, pl.ds(i * _BN, _BN)] = blk
        m0 = sm[0]
        m1 = jnp.maximum(m0, jnp.max(blkm))
        s1 = sm[1] * jnp.exp(m0 - m1) + jnp.sum(
            jnp.where(valid, jnp.exp(blkm - m1), 0.0))
        sm[0] = m1
        sm[1] = s1

        @pl.when(i == _NB - 1)
        def _():
            sm[0] = m1 + jnp.log(s1)

    @pl.when(p == 1)
    def _():
        out_ref[...] = out_s[:, pl.ds(i * _BN, _BN)] - sm[0]


def _tc_mlp(embedded, W1, b1_row, W2, b2_row):
    return pl.pallas_call(
        _mlp_body,
        grid=(2, _NB),
        in_specs=[
            pl.BlockSpec((1, 2 * _CTX * _EMBD), lambda p, i: (0, 0)),
            pl.BlockSpec((2 * _CTX * _EMBD, _HID), lambda p, i: (0, 0)),
            pl.BlockSpec((1, _HID), lambda p, i: (0, 0)),
            pl.BlockSpec((_HID, _BN),
                         lambda p, i: (0, i * (1 - p) + (_NB - 1) * p)),
            pl.BlockSpec((1, _BN),
                         lambda p, i: (0, i * (1 - p) + (_NB - 1) * p)),
        ],
        out_specs=pl.BlockSpec((1, _BN), lambda p, i: (0, i * p)),
        out_shape=jax.ShapeDtypeStruct((1, _VOCAB), jnp.float32),
        scratch_shapes=[
            pltpu.VMEM((1, _HID), jnp.float32),
            pltpu.VMEM((1, _NB * _BN), jnp.float32),
            pltpu.SMEM((2,), jnp.float32),
        ],
        compiler_params=pltpu.CompilerParams(
            dimension_semantics=("arbitrary", "arbitrary"),
        ),
    )(embedded, W1, b1_row, W2, b2_row)


def kernel(inputs, emb, W1, b1, W2, b2):
    embedded = _sc_gather(inputs, emb).reshape(1, 2 * _CTX * _EMBD)
    return _tc_mlp(embedded, W1, b1.reshape(1, _HID), W2,
                   b2.reshape(1, _VOCAB))
